# Initial kernel scaffold; baseline (speedup 1.0000x reference)
import jax
import jax.numpy as jnp
from jax.experimental import pallas as pl


def kernel(input, weight):
    raise NotImplementedError("write your pallas kernel here")


# speedup vs baseline: 1.3662x; 1.3662x over previous
import functools

import jax
import jax.numpy as jnp
from jax import lax
from jax.experimental import pallas as pl
from jax.experimental.pallas import tpu as pltpu
from jax.experimental.pallas import tpu_sc as plsc

M_TILE = 256


def _argmin_body(x_ref, w_ref, w2_ref, idx_ref):
    x = x_ref[...]
    w = w_ref[...]
    cov = lax.dot_general(
        x, w, (((1,), (1,)), ((), ())),
        preferred_element_type=jnp.float32)
    scores = w2_ref[...] - 2.0 * cov
    mn = jnp.min(scores, axis=1, keepdims=True)
    kiota = lax.broadcasted_iota(jnp.int32, scores.shape, 1)
    cand = jnp.where(scores == mn, kiota, jnp.int32(2**30))
    idx_ref[...] = jnp.min(cand, axis=1, keepdims=True)


def _gather_kernel(idx_hbm, table_hbm, val_hbm, vec_hbm, idx_v, rows_v, sem):
    info = plsc.get_sparse_core_info()
    nw = info.num_cores * info.num_subcores
    b = val_hbm.shape[0]
    ch = idx_v.shape[0]
    per_w = b // nw
    wid = lax.axis_index("s") * info.num_cores + lax.axis_index("c")
    for c in range(per_w // ch):
        base = wid * per_w + c * ch
        pltpu.sync_copy(idx_hbm.at[pl.ds(base, ch)], idx_v)
        pltpu.async_copy(table_hbm.at[idx_v], rows_v, sem).wait()
        pltpu.sync_copy(rows_v, val_hbm.at[pl.ds(base, ch)])
        pltpu.sync_copy(rows_v, vec_hbm.at[pl.ds(base, ch)])


def kernel(input, weight):
    B, T, d = input.shape
    K = weight.shape[0]
    M = B * T
    x2d = input.reshape(M, d).astype(jnp.bfloat16)
    w_bf = weight.astype(jnp.bfloat16)
    w2_row = jnp.sum(weight * weight, axis=1).reshape(1, K)

    grid = (M // M_TILE,)
    idx_col = pl.pallas_call(
        _argmin_body,
        grid=grid,
        in_specs=[
            pl.BlockSpec((M_TILE, d), lambda m: (m, 0)),
            pl.BlockSpec((K, d), lambda m: (0, 0)),
            pl.BlockSpec((1, K), lambda m: (0, 0)),
        ],
        out_specs=pl.BlockSpec((M_TILE, 1), lambda m: (m, 0)),
        out_shape=jax.ShapeDtypeStruct((M, 1), jnp.int32),
    )(x2d, w_bf, w2_row)
    idx_flat = idx_col.reshape(M)

    ch = 128
    mesh = plsc.VectorSubcoreMesh(core_axis_name="c", subcore_axis_name="s")
    gather = functools.partial(
        pl.kernel,
        out_type=(jax.ShapeDtypeStruct((M, d), jnp.float32),
                  jax.ShapeDtypeStruct((M, d), jnp.float32)),
        mesh=mesh,
        scratch_types=[
            pltpu.VMEM((ch,), jnp.int32),
            pltpu.VMEM((ch, d), jnp.float32),
            pltpu.SemaphoreType.DMA,
        ],
    )(_gather_kernel)
    values, vectors = gather(idx_flat, weight)
    values = values.reshape(B, T, d)
    vectors = vectors.reshape(B, T, d)

    indices = idx_flat.reshape(B, T)
    return (values, indices, vectors)
